# baseline (speedup 1.0000x reference)
import jax
import jax.numpy as jnp
from jax import lax
from jax.experimental import pallas as pl

B = 8
NL = 13
L = 512
D = 768
NBINS = 320
DH = D // 2
SPB = 1


def _body(vq_ref, xlo_ref, xhi_ref, o_ref):
    for i in range(SPB):
        v = vq_ref[i]
        iota_s = lax.broadcasted_iota(jnp.int32, (2 * NBINS, L), 0)
        is_x = iota_s < NBINS
        iota_mod = jnp.where(is_x, iota_s, iota_s - NBINS)
        tgt = jnp.where(is_x, v[0:1, :], v[1:2, :])
        et = (tgt == iota_mod).astype(jnp.bfloat16)
        ones_col = jnp.ones((L, 1), jnp.bfloat16)
        dn_nn = (((1,), (0,)), ((), ()))
        c = lax.dot_general(et, ones_col, dn_nn,
                            preferred_element_type=jnp.float32)
        cs = (c - 256.0).astype(jnp.bfloat16)
        dn_cc = (((0,), (0,)), ((), ()))
        g = lax.dot_general(cs, et, dn_cc,
                            preferred_element_type=jnp.float32)
        r = 1.0 / (g + 512.0)
        w = jnp.transpose(r * (1.0 / jnp.sum(r)))
        olo = jnp.sum(xlo_ref[i, 0] * w, axis=0, keepdims=True)
        ohi = jnp.sum(xhi_ref[i, 0] * w, axis=0, keepdims=True)
        o_ref[i] = jnp.concatenate([olo, ohi], axis=1)


def kernel(input_feature, input_lengths, vq_indices):
    del input_lengths
    vq = jnp.transpose(vq_indices.astype(jnp.int32), (0, 2, 1))
    out = pl.pallas_call(
        _body,
        grid=(B // SPB,),
        in_specs=[
            pl.BlockSpec((SPB, 2, L), lambda b: (b, 0, 0)),
            pl.BlockSpec((SPB, 1, L, DH), lambda b: (b, NL - 1, 0, 0)),
            pl.BlockSpec((SPB, 1, L, DH), lambda b: (b, NL - 1, 0, 1)),
        ],
        out_specs=pl.BlockSpec((SPB, 1, D), lambda b: (b, 0, 0)),
        out_shape=jax.ShapeDtypeStruct((B, 1, D), jnp.float32),
    )(vq, input_feature, input_feature)
    return out.reshape(B, D)
